# baseline (speedup 1.0000x reference)
import functools

import jax
import jax.numpy as jnp
from jax import lax
from jax.experimental import pallas as pl
from jax.experimental.pallas import tpu as pltpu
from jax.experimental.pallas import tpu_sc as plsc

E = 64
H = 329
S = 329
NW = 32
EPW = E // NW
NB = 41
RPW = EPW * NB * 8

_mesh = plsc.VectorSubcoreMesh(core_axis_name="c", subcore_axis_name="s")


@functools.partial(
    pl.kernel,
    mesh=_mesh,
    out_type=jax.ShapeDtypeStruct((NW, RPW * 16), jnp.float32),
    scratch_types=[
        pltpu.VMEM((2, 8, S), jnp.float32),
        pltpu.VMEM((336,), jnp.float32),
        pltpu.VMEM((RPW * 16,), jnp.float32),
        pltpu.SemaphoreType.DMA,
        pltpu.SemaphoreType.DMA,
    ],
)
def _sc_v(w_hbm, owp_hbm, out_hbm, wbuf, owb, acc, sem0, sem1):
    wid = lax.axis_index("s") * 2 + lax.axis_index("c")
    e0 = wid * EPW
    sems = [sem0, sem1]

    pltpu.sync_copy(owp_hbm, owb)

    def _copy(g, buf):
        e = e0 + g // NB
        r0 = pl.multiple_of(lax.rem(g, NB) * 8, 8)
        return pltpu.make_async_copy(
            w_hbm.at[e, pl.ds(r0, 8), :], wbuf.at[buf], sems[buf])

    _copy(0, 0).start()
    _copy(1, 1).start()

    def _pair(g2, carry):
        for half in (0, 1):
            g = g2 * 2 + half
            _copy(g, half).wait()
            for r in range(8):
                a = wbuf[half, r, pl.ds(0, 16)] * owb[pl.ds(0, 16)]
                for j in range(1, 20):
                    a = a + (wbuf[half, r, pl.ds(j * 16, 16)]
                             * owb[pl.ds(j * 16, 16)])
                a = a + wbuf[half, r, pl.ds(313, 16)] * owb[pl.ds(320, 16)]
                acc[pl.ds((g * 8 + r) * 16, 16)] = a

            @pl.when(g2 < NB - 1)
            def _next():
                _copy(g + 2, half).start()

        return carry

    lax.fori_loop(0, NB, _pair, 0)
    pltpu.sync_copy(acc, out_hbm.at[wid])


def _moe_tc_body(x_hbm, gw_ref, w_hbm, vsc_ref, em_ref, ow_ref, ob_ref,
                 gates_ref, y_ref, emo_ref, h_vmem, wl_vmem, sems):
    hcp = pltpu.make_async_copy(x_hbm.at[:, 0, :], h_vmem, sems.at[0])
    hcp.start()
    lcp = pltpu.make_async_copy(
        w_hbm.at[:, pl.ds(H - 1, 1), :], wl_vmem, sems.at[1])
    lcp.start()

    ow = ow_ref[...]
    b = ob_ref[0, 0]

    vmin = jax.lax.dot_general(
        em_ref[...], ow, (((1,), (1,)), ((), ())),
        preferred_element_type=jnp.float32)

    lcp.wait()
    vlast = jax.lax.dot_general(
        wl_vmem[:, 0, :], ow, (((1,), (1,)), ((), ())),
        preferred_element_type=jnp.float32)

    E_, HS = vsc_ref.shape
    vsc = jnp.sum(vsc_ref[...].reshape(E_, HS // 16, 16), axis=2)

    hcp.wait()
    h = h_vmem[...]

    gates_ref[...] = jax.lax.dot_general(
        h, gw_ref[...], (((1,), (1,)), ((), ())),
        preferred_element_type=jnp.float32)
    emo_ref[...] = jax.lax.dot_general(
        h, vmin, (((1,), (0,)), ((), ()))) + b

    y_ref[...] = (
        jax.lax.dot_general(
            h[:, :H - 1], vsc, (((1,), (1,)), ((), ())),
            preferred_element_type=jnp.float32)
        + jax.lax.dot_general(
            h[:, H - 1:], vlast, (((1,), (1,)), ((), ())),
            preferred_element_type=jnp.float32)
        + b)


def kernel(x, gate_weights, expert_weights, expert_min, out_w, out_b):
    B = x.shape[0]
    ob2 = out_b.reshape(1, 1)

    ow1 = out_w[0]
    owp = jnp.concatenate([ow1[:320], jnp.zeros((7,), jnp.float32),
                           ow1[320:]])

    vw = _sc_v(expert_weights, owp)
    vp = vw.reshape(E, (H - 1) * 16)

    gates, y2, emo = pl.pallas_call(
        _moe_tc_body,
        in_specs=[
            pl.BlockSpec(memory_space=pltpu.MemorySpace.HBM),
            pl.BlockSpec(memory_space=pltpu.VMEM),
            pl.BlockSpec(memory_space=pltpu.MemorySpace.HBM),
            pl.BlockSpec(memory_space=pltpu.VMEM),
            pl.BlockSpec(memory_space=pltpu.VMEM),
            pl.BlockSpec(memory_space=pltpu.VMEM),
            pl.BlockSpec(memory_space=pltpu.VMEM),
        ],
        out_shape=[
            jax.ShapeDtypeStruct((B, E), jnp.float32),
            jax.ShapeDtypeStruct((B, E), jnp.float32),
            jax.ShapeDtypeStruct((B, 1), jnp.float32),
        ],
        scratch_shapes=[
            pltpu.VMEM((B, H), jnp.float32),
            pltpu.VMEM((E, 1, S), jnp.float32),
            pltpu.SemaphoreType.DMA((2,)),
        ],
    )(x, gate_weights, expert_weights, vp, expert_min, out_w, ob2)

    return (gates, y2.reshape(B, E, 1), emo)


# speedup vs baseline: 1.8234x; 1.8234x over previous
import jax
import jax.numpy as jnp
from jax.experimental import pallas as pl
from jax.experimental.pallas import tpu as pltpu

NCHUNK = 8


def _moe_body(x_hbm, gw_ref, w_hbm, em_ref, ow_ref, ob_ref,
              gates_ref, y_ref, emo_ref, h_vmem, w_vmem, sems):
    E = w_vmem.shape[0]
    ce = E // NCHUNK
    hcp = pltpu.make_async_copy(x_hbm.at[:, 0, :], h_vmem, sems.at[NCHUNK])
    hcp.start()
    hcp.wait()

    wcopies = [
        pltpu.make_async_copy(
            w_hbm.at[pl.ds(k * ce, ce)], w_vmem.at[pl.ds(k * ce, ce)],
            sems.at[k])
        for k in range(NCHUNK)
    ]
    for c in wcopies:
        c.start()

    ow = ow_ref[...]
    b = ob_ref[0, 0]
    h = h_vmem[...]

    gates_ref[...] = jax.lax.dot_general(
        h, gw_ref[...], (((1,), (1,)), ((), ())),
        preferred_element_type=jnp.float32)

    vmin = jax.lax.dot_general(
        em_ref[...], ow, (((1,), (1,)), ((), ())),
        preferred_element_type=jnp.float32)
    emo_ref[...] = jax.lax.dot_general(
        h, vmin, (((1,), (0,)), ((), ()))) + b

    vparts = []
    for k, c in enumerate(wcopies):
        c.wait()
        vparts.append(
            jnp.sum(w_vmem[pl.ds(k * ce, ce)] * ow[None, :, :], axis=2))
    v = jnp.concatenate(vparts, axis=0)

    y_ref[...] = jax.lax.dot_general(
        h, v, (((1,), (1,)), ((), ())),
        preferred_element_type=jnp.float32) + b


def kernel(x, gate_weights, expert_weights, expert_min, out_w, out_b):
    B, _, H = x.shape
    E, _, S = expert_weights.shape
    ob2 = out_b.reshape(1, 1)

    gates, y2, emo = pl.pallas_call(
        _moe_body,
        in_specs=[
            pl.BlockSpec(memory_space=pltpu.MemorySpace.HBM),
            pl.BlockSpec(memory_space=pltpu.VMEM),
            pl.BlockSpec(memory_space=pltpu.MemorySpace.HBM),
            pl.BlockSpec(memory_space=pltpu.VMEM),
            pl.BlockSpec(memory_space=pltpu.VMEM),
            pl.BlockSpec(memory_space=pltpu.VMEM),
        ],
        out_shape=[
            jax.ShapeDtypeStruct((B, E), jnp.float32),
            jax.ShapeDtypeStruct((B, E), jnp.float32),
            jax.ShapeDtypeStruct((B, 1), jnp.float32),
        ],
        scratch_shapes=[
            pltpu.VMEM((B, H), jnp.float32),
            pltpu.VMEM((E, H, S), jnp.float32),
            pltpu.SemaphoreType.DMA((NCHUNK + 1,)),
        ],
    )(x, gate_weights, expert_weights, expert_min, out_w, ob2)

    return (gates, y2.reshape(B, E, 1), emo)
